# speedup vs baseline: 1.1206x; 1.1206x over previous
import functools

import jax
import jax.numpy as jnp
from jax import lax
from jax.experimental import pallas as pl
from jax.experimental.pallas import tpu as pltpu
from jax.experimental.pallas import tpu_sc as plsc


def _argmin_body(a_ref, z2_ref, cb_ref, cn_ref, ids_ref, ind_ref, loss_ref,
                 bv_scr, bi_scr):
    i = pl.program_id(0)
    j = pl.program_id(1)
    nj = pl.num_programs(1)
    mm2 = jnp.dot(z2_ref[...], cb_ref[...], preferred_element_type=jnp.float32)
    dist = (a_ref[...] - mm2) + cn_ref[...]
    bmin = jnp.min(dist, axis=1, keepdims=True)
    big = jnp.int32(2**30)
    bidx = jnp.min(jnp.where(dist == bmin, ids_ref[...], big), axis=1,
                   keepdims=True)

    @pl.when(j == 0)
    def _():
        bv_scr[...] = bmin
        bi_scr[...] = bidx

    @pl.when(j > 0)
    def _():
        upd = bmin < bv_scr[...]
        bv_scr[...] = jnp.where(upd, bmin, bv_scr[...])
        bi_scr[...] = jnp.where(upd, bidx, bi_scr[...])

    @pl.when(j == nj - 1)
    def _():
        ind_ref[...] = bi_scr[...]
        tot = jnp.sum(bv_scr[...])

        @pl.when(i == 0)
        def _():
            loss_ref[0, 0] = tot

        @pl.when(i > 0)
        def _():
            loss_ref[0, 0] = loss_ref[0, 0] + tot


def _argmin_call(a, z2, cb, cn, ids, br, bc):
    n, d = z2.shape
    e = cb.shape[1]
    grid = (n // br, e // bc)
    return pl.pallas_call(
        _argmin_body,
        grid=grid,
        in_specs=[
            pl.BlockSpec((br, 1), lambda i, j: (i, 0)),
            pl.BlockSpec((br, d), lambda i, j: (i, 0)),
            pl.BlockSpec((d, bc), lambda i, j: (0, j)),
            pl.BlockSpec((1, bc), lambda i, j: (0, j)),
            pl.BlockSpec((1, bc), lambda i, j: (0, j)),
        ],
        out_specs=[
            pl.BlockSpec((br, 1), lambda i, j: (i, 0)),
            pl.BlockSpec((1, 1), lambda i, j: (0, 0),
                         memory_space=pltpu.SMEM),
        ],
        out_shape=[
            jax.ShapeDtypeStruct((n, 1), jnp.int32),
            jax.ShapeDtypeStruct((1, 1), jnp.float32),
        ],
        scratch_shapes=[
            pltpu.VMEM((br, 1), jnp.float32),
            pltpu.VMEM((br, 1), jnp.int32),
        ],
    )(a, z2, cb, cn, ids)


def _gather_body(nc, ch, nchunk, table_hbm, idx_hbm, out_hbm, idx_v, rows_v,
                 sem):
    wid = lax.axis_index("s") * nc + lax.axis_index("c")
    base = wid * (ch * nchunk)
    for t in range(nchunk):
        off = base + t * ch
        pltpu.sync_copy(idx_hbm.at[pl.ds(off, ch)], idx_v)
        pltpu.async_copy(table_hbm.at[idx_v], rows_v, sem).wait()
        pltpu.sync_copy(rows_v, out_hbm.at[pl.ds(off, ch)])


def _gather_call(table, idx):
    v, d = table.shape
    b = idx.shape[0]
    info = plsc.get_sparse_core_info()
    nw = info.num_cores * info.num_subcores
    bpw = b // nw
    ch = min(bpw, 128)
    nchunk = bpw // ch
    mesh = plsc.VectorSubcoreMesh(core_axis_name="c", subcore_axis_name="s")
    k = functools.partial(
        pl.kernel,
        mesh=mesh,
        out_type=jax.ShapeDtypeStruct((b, d), jnp.float32),
        scratch_types=[
            pltpu.VMEM((ch,), jnp.int32),
            pltpu.VMEM((ch, d), jnp.float32),
            pltpu.SemaphoreType.DMA,
        ],
    )(functools.partial(_gather_body, info.num_cores, ch, nchunk))
    return k(table, idx)


def kernel(latents, codebook):
    n, d = latents.shape
    a = jnp.sum(latents ** 2, axis=1, keepdims=True)
    cn = jnp.sum(codebook ** 2, axis=0, keepdims=True)
    z2 = latents + latents
    e = codebook.shape[1]
    ids = lax.broadcasted_iota(jnp.int32, (1, e), 1)
    ind2d, loss_sum = _argmin_call(a, z2, codebook, cn, ids,
                                   br=min(1024, n), bc=min(1024, e))
    ind = ind2d.reshape(n)
    qraw = _gather_call(codebook.T, ind)
    quantize = latents + lax.stop_gradient(qraw - latents)
    vq_loss = loss_sum[0, 0] / jnp.float32(n * d)
    return quantize, vq_loss, ind
